# baseline (speedup 1.0000x reference)
import math

import jax
import jax.numpy as jnp
from jax import lax
from jax.experimental import pallas as pl
from jax.experimental.pallas import tpu as pltpu
from jax.experimental.pallas import tpu_sc as plsc

B, L, D, H, V = 1024, 200, 256, 8, 100000
DH = D // H
NW = 32
CHUNK = 320
NC = 1
BC = B // NC


def _gather_rows_sc(table, idx):
    n = idx.shape[0]
    per_w = n // NW
    iters = per_w // CHUNK
    mesh = plsc.VectorSubcoreMesh(core_axis_name="c", subcore_axis_name="s")

    def body(table_ref, idx_ref, out_ref, idx_v, rows_v, sem):
        wid = lax.axis_index("s") * 2 + lax.axis_index("c")
        base = wid * per_w

        def step(i, carry):
            off = base + i * CHUNK
            pltpu.sync_copy(idx_ref.at[pl.ds(off, CHUNK)], idx_v)
            pltpu.async_copy(table_ref.at[idx_v], rows_v, sem).wait()
            pltpu.sync_copy(rows_v, out_ref.at[pl.ds(off, CHUNK)])
            return carry

        lax.fori_loop(0, iters, step, 0)

    fn = pl.kernel(
        body,
        out_type=jax.ShapeDtypeStruct((n, D), jnp.float32),
        mesh=mesh,
        scratch_types=[
            pltpu.VMEM((CHUNK,), jnp.int32),
            pltpu.VMEM((CHUNK, D), jnp.float32),
            pltpu.SemaphoreType.DMA,
        ],
    )
    return fn(table, idx)


def _ln_rows(x, g, b):
    m = jnp.mean(x, axis=-1, keepdims=True)
    xc = x - m
    v = jnp.mean(xc * xc, axis=-1, keepdims=True)
    return xc * lax.rsqrt(v + 1e-5) * g + b


BB = 8


def _transformer_body(he_ref, ne_ref, af_ref, nb_ref, ones_ref, Pk_ref,
                      Pv_ref, cdk_ref, cdv_ref, Ak_ref, Av_ref, W3_ref,
                      Aq_ref, bq_ref, Wo_ref, bo_ref, F0w_ref, F1w_ref,
                      F0b_ref, F1b_ref, g2_ref, b2_ref, g3_ref, b3_ref,
                      W4_ref, b4_ref, out_ref):
    bf = jnp.bfloat16
    f32 = jnp.float32

    hef = he_ref[...].reshape(BB * L, D).astype(bf)
    nef = ne_ref[...].reshape(BB * L, D).astype(bf)
    af3 = af_ref[...].astype(bf)
    nb = nb_ref[...]
    ones_c = ones_ref[...]

    k3 = (jnp.dot(hef, Ak_ref[...],
                  preferred_element_type=f32).astype(bf).reshape(BB, L, D)
          + Pk_ref[...][None] + af3 * cdk_ref[...])
    v3 = (jnp.dot(hef, Av_ref[...],
                  preferred_element_type=f32).astype(bf).reshape(BB, L, D)
          + Pv_ref[...][None] + af3 * cdv_ref[...])
    query = jnp.dot(nef, W3_ref[...], preferred_element_type=f32)
    q3 = (jnp.dot(nef, Aq_ref[...],
                  preferred_element_type=f32).astype(bf).reshape(BB, L, D)
          + bq_ref[...])

    ctx_rows = []
    for b in range(BB):
        q2, k2, v2 = q3[b], k3[b], v3[b]
        heads = []
        for h in range(H):
            sl = slice(h * DH, (h + 1) * DH)
            s = lax.dot_general(q2[:, sl], k2[:, sl],
                                (((1,), (1,)), ((), ())),
                                preferred_element_type=f32)
            e = jnp.exp(s).astype(bf) * nb
            vaug = jnp.concatenate([v2[:, sl], ones_c], axis=1)
            o = jnp.dot(e, vaug, preferred_element_type=f32)
            heads.append(o[:, :DH] * (1.0 / o[:, DH:DH + 1]))
        ctx_rows.append(jnp.concatenate(heads, axis=1))
    ctx = jnp.concatenate(ctx_rows, axis=0).astype(bf)

    atn = (jnp.dot(ctx, Wo_ref[...], preferred_element_type=f32)
           + bo_ref[...] + query)
    atn = _ln_rows(atn, g2_ref[...], b2_ref[...])
    hdn = jnp.maximum(
        jnp.dot(atn.astype(bf), F0w_ref[...], preferred_element_type=f32)
        + F0b_ref[...], 0.0).astype(bf)
    ffn = jnp.dot(hdn, F1w_ref[...], preferred_element_type=f32) + F1b_ref[...]
    ffn = _ln_rows(ffn + atn, g3_ref[...], b3_ref[...])
    pred = (jnp.dot(ffn.astype(bf), W4_ref[...], preferred_element_type=f32)
            + b4_ref[...])
    out_ref[...] = pred.reshape(BB, L, 1)


def _transformer_tc(he, ne, ansf, nbias, ones_c, Pk, Pv, cdk, cdv, Ak, Av,
                    W3, Aq, bq, Wo, bo, F0w, F1w, F0b, F1b, g2, b2, g3, b3,
                    W4, b4):
    def blk(shape, imap):
        return pl.BlockSpec(shape, imap)

    row = lambda i: (i, 0, 0)
    row_ne = lambda i: (i, 1, 0)
    const2 = lambda i: (0, 0)
    in_specs = [
        blk((BB, L, D), row),
        blk((BB, L, D), row_ne),
        blk((BB, L, 1), row),
        blk((L, L), const2),
        blk((L, 1), const2),
        blk((L, D), const2),
        blk((L, D), const2),
        blk((1, D), const2),
        blk((1, D), const2),
        blk((D, D), const2),
        blk((D, D), const2),
        blk((D, D), const2),
        blk((D, D), const2),
        blk((1, D), const2),
        blk((D, D), const2),
        blk((1, D), const2),
        blk((D, D), const2),
        blk((D, D), const2),
        blk((1, D), const2),
        blk((1, D), const2),
        blk((1, D), const2),
        blk((1, D), const2),
        blk((1, D), const2),
        blk((1, D), const2),
        blk((D, 1), const2),
        blk((1, 1), const2),
    ]
    out = pl.pallas_call(
        _transformer_body,
        grid=(BC // BB,),
        in_specs=in_specs,
        out_specs=pl.BlockSpec((BB, L, 1), row),
        out_shape=jax.ShapeDtypeStruct((BC, L, 1), jnp.float32),
        compiler_params=pltpu.CompilerParams(
            dimension_semantics=("arbitrary",)),
    )(he, ne, ansf, nbias, ones_c, Pk, Pv, cdk, cdv, Ak, Av, W3, Aq, bq, Wo,
      bo, F0w, F1w, F0b, F1b, g2, b2, g3, b3, W4, b4)
    return out


def kernel(hist_seq, hist_answers, new_seq, node_emb, corr_emb, pos_emb, W0,
           b0, W1, W2, W3, Wq, bq, Wk, bk, Wv, bv, Wo, bo, F0w, F0b, F1w,
           F1b, ln2g, ln2b, ln3g, ln3b, W4, b4):
    idx_arr = jnp.concatenate([hist_seq, new_seq], axis=1).astype(jnp.int32)

    bf = jnp.bfloat16
    corrW = corr_emb @ W0[D:]
    P = pos_emb + b0[None, :] + corrW[0][None, :]
    cdelta = (corrW[1] - corrW[0]).reshape(1, D)
    scale = jnp.float32(1.0 / math.sqrt(DH))
    cols_gt_rows = (jnp.arange(L)[None, :] > jnp.arange(L)[:, None])
    nbias = jnp.where(cols_gt_rows, 0.0, 1.0).astype(bf)
    ones_c = jnp.ones((L, 1), bf)
    Wkf = W2 @ Wk
    Wvf = W1 @ Wv
    Ak = (W0[:D] @ Wkf).astype(bf)
    Av = (W0[:D] @ Wvf).astype(bf)
    Pk = (P @ Wkf + bk[None, :]).astype(bf)
    Pv = (P @ Wvf + bv[None, :]).astype(bf)
    cdk = (cdelta @ Wkf).astype(bf)
    cdv = (cdelta @ Wvf).astype(bf)
    W3b = W3.astype(bf)
    Aq = ((W3 @ Wq) * scale).astype(bf)
    Wob = Wo.astype(bf)
    F0wb = F0w.astype(bf)
    F1wb = F1w.astype(bf)
    W4b = W4.astype(bf)
    ansf = hist_answers.reshape(B, L, 1)

    outs = []
    for c in range(NC):
        bsl = slice(c * BC, (c + 1) * BC)
        hn = _gather_rows_sc(
            node_emb, idx_arr[bsl].reshape(-1)).reshape(BC, 2 * L, D)
        outs.append(_transformer_tc(
            hn, hn, ansf[bsl], nbias, ones_c, Pk, Pv, cdk, cdv, Ak, Av,
            W3b, Aq, (bq * scale).reshape(1, D).astype(bf),
            Wob, bo.reshape(1, D),
            F0wb, F1wb, F0b.reshape(1, D), F1b.reshape(1, D),
            ln2g.reshape(1, D), ln2b.reshape(1, D), ln3g.reshape(1, D),
            ln3b.reshape(1, D), W4b, b4.reshape(1, 1)))
    return jnp.concatenate(outs, axis=0).reshape(B, L)


# speedup vs baseline: 1.1237x; 1.1237x over previous
import math

import jax
import jax.numpy as jnp
from jax import lax
from jax.experimental import pallas as pl
from jax.experimental.pallas import tpu as pltpu
from jax.experimental.pallas import tpu_sc as plsc

B, L, D, H, V = 1024, 200, 256, 8, 100000
DH = D // H
NW = 32
CHUNK = 320
NC = 2
BC = B // NC


def _gather_rows_sc(table, idx):
    n = idx.shape[0]
    per_w = n // NW
    iters = per_w // CHUNK
    mesh = plsc.VectorSubcoreMesh(core_axis_name="c", subcore_axis_name="s")

    def body(table_ref, idx_ref, out_ref, idx_v, rows_v, sem):
        wid = lax.axis_index("s") * 2 + lax.axis_index("c")
        base = wid * per_w

        def step(i, carry):
            off = base + i * CHUNK
            pltpu.sync_copy(idx_ref.at[pl.ds(off, CHUNK)], idx_v)
            pltpu.async_copy(table_ref.at[idx_v], rows_v, sem).wait()
            pltpu.sync_copy(rows_v, out_ref.at[pl.ds(off, CHUNK)])
            return carry

        lax.fori_loop(0, iters, step, 0)

    fn = pl.kernel(
        body,
        out_type=jax.ShapeDtypeStruct((n, D), jnp.float32),
        mesh=mesh,
        scratch_types=[
            pltpu.VMEM((CHUNK,), jnp.int32),
            pltpu.VMEM((CHUNK, D), jnp.float32),
            pltpu.SemaphoreType.DMA,
        ],
    )
    return fn(table, idx)


def _ln_rows(x, g, b):
    m = jnp.mean(x, axis=-1, keepdims=True)
    xc = x - m
    v = jnp.mean(xc * xc, axis=-1, keepdims=True)
    return xc * lax.rsqrt(v + 1e-5) * g + b


BB = 8


def _transformer_body(he_ref, ne_ref, af_ref, nb_ref, ones_ref, Pk_ref,
                      Pv_ref, cdk_ref, cdv_ref, Ak_ref, Av_ref, W3_ref,
                      Aq_ref, bq_ref, Wo_ref, bo_ref, F0w_ref, F1w_ref,
                      F0b_ref, F1b_ref, g2_ref, b2_ref, g3_ref, b3_ref,
                      W4_ref, b4_ref, out_ref):
    bf = jnp.bfloat16
    f32 = jnp.float32

    hef = he_ref[...].reshape(BB * L, D).astype(bf)
    nef = ne_ref[...].reshape(BB * L, D).astype(bf)
    af3 = af_ref[...].astype(bf)
    nb = nb_ref[...]
    ones_c = ones_ref[...]

    k3 = (jnp.dot(hef, Ak_ref[...],
                  preferred_element_type=f32).astype(bf).reshape(BB, L, D)
          + Pk_ref[...][None] + af3 * cdk_ref[...])
    v3 = (jnp.dot(hef, Av_ref[...],
                  preferred_element_type=f32).astype(bf).reshape(BB, L, D)
          + Pv_ref[...][None] + af3 * cdv_ref[...])
    query = jnp.dot(nef, W3_ref[...], preferred_element_type=f32)
    q3 = (jnp.dot(nef, Aq_ref[...],
                  preferred_element_type=f32).astype(bf).reshape(BB, L, D)
          + bq_ref[...])

    ctx_rows = []
    for b in range(BB):
        q2, k2, v2 = q3[b], k3[b], v3[b]
        heads = []
        for h in range(H):
            sl = slice(h * DH, (h + 1) * DH)
            s = lax.dot_general(q2[:, sl], k2[:, sl],
                                (((1,), (1,)), ((), ())),
                                preferred_element_type=f32)
            e = jnp.exp(s).astype(bf) * nb
            vaug = jnp.concatenate([v2[:, sl], ones_c], axis=1)
            o = jnp.dot(e, vaug, preferred_element_type=f32)
            heads.append(o[:, :DH] * (1.0 / o[:, DH:DH + 1]))
        ctx_rows.append(jnp.concatenate(heads, axis=1))
    ctx = jnp.concatenate(ctx_rows, axis=0).astype(bf)

    atn = (jnp.dot(ctx, Wo_ref[...], preferred_element_type=f32)
           + bo_ref[...] + query)
    atn = _ln_rows(atn, g2_ref[...], b2_ref[...])
    hdn = jnp.maximum(
        jnp.dot(atn.astype(bf), F0w_ref[...], preferred_element_type=f32)
        + F0b_ref[...], 0.0).astype(bf)
    ffn = jnp.dot(hdn, F1w_ref[...], preferred_element_type=f32) + F1b_ref[...]
    ffn = _ln_rows(ffn + atn, g3_ref[...], b3_ref[...])
    pred = (jnp.dot(ffn.astype(bf), W4_ref[...], preferred_element_type=f32)
            + b4_ref[...])
    out_ref[...] = pred.reshape(BB, L, 1)


def _transformer_tc(he, ne, ansf, nbias, ones_c, Pk, Pv, cdk, cdv, Ak, Av,
                    W3, Aq, bq, Wo, bo, F0w, F1w, F0b, F1b, g2, b2, g3, b3,
                    W4, b4):
    def blk(shape, imap):
        return pl.BlockSpec(shape, imap)

    row = lambda i: (i, 0, 0)
    row_ne = lambda i: (i, 1, 0)
    const2 = lambda i: (0, 0)
    in_specs = [
        blk((BB, L, D), row),
        blk((BB, L, D), row_ne),
        blk((BB, L, 1), row),
        blk((L, L), const2),
        blk((L, 1), const2),
        blk((L, D), const2),
        blk((L, D), const2),
        blk((1, D), const2),
        blk((1, D), const2),
        blk((D, D), const2),
        blk((D, D), const2),
        blk((D, D), const2),
        blk((D, D), const2),
        blk((1, D), const2),
        blk((D, D), const2),
        blk((1, D), const2),
        blk((D, D), const2),
        blk((D, D), const2),
        blk((1, D), const2),
        blk((1, D), const2),
        blk((1, D), const2),
        blk((1, D), const2),
        blk((1, D), const2),
        blk((1, D), const2),
        blk((D, 1), const2),
        blk((1, 1), const2),
    ]
    out = pl.pallas_call(
        _transformer_body,
        grid=(BC // BB,),
        in_specs=in_specs,
        out_specs=pl.BlockSpec((BB, L, 1), row),
        out_shape=jax.ShapeDtypeStruct((BC, L, 1), jnp.float32),
        compiler_params=pltpu.CompilerParams(
            dimension_semantics=("arbitrary",)),
    )(he, ne, ansf, nbias, ones_c, Pk, Pv, cdk, cdv, Ak, Av, W3, Aq, bq, Wo,
      bo, F0w, F1w, F0b, F1b, g2, b2, g3, b3, W4, b4)
    return out


def kernel(hist_seq, hist_answers, new_seq, node_emb, corr_emb, pos_emb, W0,
           b0, W1, W2, W3, Wq, bq, Wk, bk, Wv, bv, Wo, bo, F0w, F0b, F1w,
           F1b, ln2g, ln2b, ln3g, ln3b, W4, b4):
    idx_arr = jnp.concatenate([hist_seq, new_seq], axis=1).astype(jnp.int32)

    bf = jnp.bfloat16
    corrW = corr_emb @ W0[D:]
    P = pos_emb + b0[None, :] + corrW[0][None, :]
    cdelta = (corrW[1] - corrW[0]).reshape(1, D)
    scale = jnp.float32(1.0 / math.sqrt(DH))
    cols_gt_rows = (jnp.arange(L)[None, :] > jnp.arange(L)[:, None])
    nbias = jnp.where(cols_gt_rows, 0.0, 1.0).astype(bf)
    ones_c = jnp.ones((L, 1), bf)
    Wkf = W2 @ Wk
    Wvf = W1 @ Wv
    Ak = (W0[:D] @ Wkf).astype(bf)
    Av = (W0[:D] @ Wvf).astype(bf)
    Pk = (P @ Wkf + bk[None, :]).astype(bf)
    Pv = (P @ Wvf + bv[None, :]).astype(bf)
    cdk = (cdelta @ Wkf).astype(bf)
    cdv = (cdelta @ Wvf).astype(bf)
    W3b = W3.astype(bf)
    Aq = ((W3 @ Wq) * scale).astype(bf)
    Wob = Wo.astype(bf)
    F0wb = F0w.astype(bf)
    F1wb = F1w.astype(bf)
    W4b = W4.astype(bf)
    ansf = hist_answers.reshape(B, L, 1)

    outs = []
    for c in range(NC):
        bsl = slice(c * BC, (c + 1) * BC)
        hn = _gather_rows_sc(
            node_emb, idx_arr[bsl].reshape(-1)).reshape(BC, 2 * L, D)
        outs.append(_transformer_tc(
            hn, hn, ansf[bsl], nbias, ones_c, Pk, Pv, cdk, cdv, Ak, Av,
            W3b, Aq, (bq * scale).reshape(1, D).astype(bf),
            Wob, bo.reshape(1, D),
            F0wb, F1wb, F0b.reshape(1, D), F1b.reshape(1, D),
            ln2g.reshape(1, D), ln2b.reshape(1, D), ln3g.reshape(1, D),
            ln3b.reshape(1, D), W4b, b4.reshape(1, 1)))
    return jnp.concatenate(outs, axis=0).reshape(B, L)


# speedup vs baseline: 1.1869x; 1.0562x over previous
import math

import jax
import jax.numpy as jnp
from jax import lax
from jax.experimental import pallas as pl
from jax.experimental.pallas import tpu as pltpu
from jax.experimental.pallas import tpu_sc as plsc

B, L, D, H, V = 1024, 200, 256, 8, 100000
DH = D // H
NW = 32
CHUNK = 320
NC = 4
BC = B // NC


def _gather_rows_sc(table, idx):
    n = idx.shape[0]
    per_w = n // NW
    iters = per_w // CHUNK
    mesh = plsc.VectorSubcoreMesh(core_axis_name="c", subcore_axis_name="s")

    def body(table_ref, idx_ref, out_ref, idx_v, rows_v, sem):
        wid = lax.axis_index("s") * 2 + lax.axis_index("c")
        base = wid * per_w

        def step(i, carry):
            off = base + i * CHUNK
            pltpu.sync_copy(idx_ref.at[pl.ds(off, CHUNK)], idx_v)
            pltpu.async_copy(table_ref.at[idx_v], rows_v, sem).wait()
            pltpu.sync_copy(rows_v, out_ref.at[pl.ds(off, CHUNK)])
            return carry

        lax.fori_loop(0, iters, step, 0)

    fn = pl.kernel(
        body,
        out_type=jax.ShapeDtypeStruct((n, D), jnp.float32),
        mesh=mesh,
        scratch_types=[
            pltpu.VMEM((CHUNK,), jnp.int32),
            pltpu.VMEM((CHUNK, D), jnp.float32),
            pltpu.SemaphoreType.DMA,
        ],
    )
    return fn(table, idx)


def _ln_rows(x, g, b):
    m = jnp.mean(x, axis=-1, keepdims=True)
    xc = x - m
    v = jnp.mean(xc * xc, axis=-1, keepdims=True)
    return xc * lax.rsqrt(v + 1e-5) * g + b


BB = 8


def _transformer_body(he_ref, ne_ref, af_ref, nb_ref, ones_ref, Pk_ref,
                      Pv_ref, cdk_ref, cdv_ref, Ak_ref, Av_ref, W3_ref,
                      Aq_ref, bq_ref, Wo_ref, bo_ref, F0w_ref, F1w_ref,
                      F0b_ref, F1b_ref, g2_ref, b2_ref, g3_ref, b3_ref,
                      W4_ref, b4_ref, out_ref):
    bf = jnp.bfloat16
    f32 = jnp.float32

    hef = he_ref[...].reshape(BB * L, D).astype(bf)
    nef = ne_ref[...].reshape(BB * L, D).astype(bf)
    af3 = af_ref[...].astype(bf).reshape(BB, L, 1)
    nb = nb_ref[...]
    ones_c = ones_ref[...]

    k3 = (jnp.dot(hef, Ak_ref[...],
                  preferred_element_type=f32).astype(bf).reshape(BB, L, D)
          + Pk_ref[...][None] + af3 * cdk_ref[...])
    v3 = (jnp.dot(hef, Av_ref[...],
                  preferred_element_type=f32).astype(bf).reshape(BB, L, D)
          + Pv_ref[...][None] + af3 * cdv_ref[...])
    query = jnp.dot(nef, W3_ref[...], preferred_element_type=f32)
    q3 = (jnp.dot(nef, Aq_ref[...],
                  preferred_element_type=f32).astype(bf).reshape(BB, L, D)
          + bq_ref[...])

    ctx_rows = []
    for b in range(BB):
        q2, k2, v2 = q3[b], k3[b], v3[b]
        heads = []
        for h in range(H):
            sl = slice(h * DH, (h + 1) * DH)
            s = lax.dot_general(q2[:, sl], k2[:, sl],
                                (((1,), (1,)), ((), ())),
                                preferred_element_type=f32)
            e = jnp.exp(s).astype(bf) * nb
            vaug = jnp.concatenate([v2[:, sl], ones_c], axis=1)
            o = jnp.dot(e, vaug, preferred_element_type=f32)
            heads.append(o[:, :DH] * (1.0 / o[:, DH:DH + 1]))
        ctx_rows.append(jnp.concatenate(heads, axis=1))
    ctx = jnp.concatenate(ctx_rows, axis=0).astype(bf)

    atn = (jnp.dot(ctx, Wo_ref[...], preferred_element_type=f32)
           + bo_ref[...] + query)
    atn = _ln_rows(atn, g2_ref[...], b2_ref[...])
    hdn = jnp.maximum(
        jnp.dot(atn.astype(bf), F0w_ref[...], preferred_element_type=f32)
        + F0b_ref[...], 0.0).astype(bf)
    ffn = jnp.dot(hdn, F1w_ref[...], preferred_element_type=f32) + F1b_ref[...]
    ffn = _ln_rows(ffn + atn, g3_ref[...], b3_ref[...])
    pred = (jnp.dot(ffn.astype(bf), W4_ref[...], preferred_element_type=f32)
            + b4_ref[...])
    out_ref[...] = pred.reshape(BB, L)


def _transformer_tc(he, ne, ansf, nbias, ones_c, Pk, Pv, cdk, cdv, Ak, Av,
                    W3, Aq, bq, Wo, bo, F0w, F1w, F0b, F1b, g2, b2, g3, b3,
                    W4, b4):
    def blk(shape, imap):
        return pl.BlockSpec(shape, imap)

    row = lambda i: (i, 0, 0)
    row_ne = lambda i: (i, 1, 0)
    const2 = lambda i: (0, 0)
    in_specs = [
        blk((BB, L, D), row),
        blk((BB, L, D), row_ne),
        blk((BB, L), lambda i: (i, 0)),
        blk((L, L), const2),
        blk((L, 1), const2),
        blk((L, D), const2),
        blk((L, D), const2),
        blk((1, D), const2),
        blk((1, D), const2),
        blk((D, D), const2),
        blk((D, D), const2),
        blk((D, D), const2),
        blk((D, D), const2),
        blk((1, D), const2),
        blk((D, D), const2),
        blk((1, D), const2),
        blk((D, D), const2),
        blk((D, D), const2),
        blk((1, D), const2),
        blk((1, D), const2),
        blk((1, D), const2),
        blk((1, D), const2),
        blk((1, D), const2),
        blk((1, D), const2),
        blk((D, 1), const2),
        blk((1, 1), const2),
    ]
    out = pl.pallas_call(
        _transformer_body,
        grid=(BC // BB,),
        in_specs=in_specs,
        out_specs=pl.BlockSpec((BB, L), lambda i: (i, 0)),
        out_shape=jax.ShapeDtypeStruct((BC, L), jnp.float32),
        compiler_params=pltpu.CompilerParams(
            dimension_semantics=("arbitrary",)),
    )(he, ne, ansf, nbias, ones_c, Pk, Pv, cdk, cdv, Ak, Av, W3, Aq, bq, Wo,
      bo, F0w, F1w, F0b, F1b, g2, b2, g3, b3, W4, b4)
    return out


def kernel(hist_seq, hist_answers, new_seq, node_emb, corr_emb, pos_emb, W0,
           b0, W1, W2, W3, Wq, bq, Wk, bk, Wv, bv, Wo, bo, F0w, F0b, F1w,
           F1b, ln2g, ln2b, ln3g, ln3b, W4, b4):
    idx_arr = jnp.concatenate([hist_seq, new_seq], axis=1).astype(jnp.int32)

    bf = jnp.bfloat16
    corrW = corr_emb @ W0[D:]
    P = pos_emb + b0[None, :] + corrW[0][None, :]
    cdelta = (corrW[1] - corrW[0]).reshape(1, D)
    scale = jnp.float32(1.0 / math.sqrt(DH))
    cols_gt_rows = (jnp.arange(L)[None, :] > jnp.arange(L)[:, None])
    nbias = jnp.where(cols_gt_rows, 0.0, 1.0).astype(bf)
    ones_c = jnp.ones((L, 1), bf)
    Wkf = W2 @ Wk
    Wvf = W1 @ Wv
    Ak = (W0[:D] @ Wkf).astype(bf)
    Av = (W0[:D] @ Wvf).astype(bf)
    Pk = (P @ Wkf + bk[None, :]).astype(bf)
    Pv = (P @ Wvf + bv[None, :]).astype(bf)
    cdk = (cdelta @ Wkf).astype(bf)
    cdv = (cdelta @ Wvf).astype(bf)
    W3b = W3.astype(bf)
    Aq = ((W3 @ Wq) * scale).astype(bf)
    Wob = Wo.astype(bf)
    F0wb = F0w.astype(bf)
    F1wb = F1w.astype(bf)
    W4b = W4.astype(bf)
    ansf = hist_answers

    outs = []
    for c in range(NC):
        bsl = slice(c * BC, (c + 1) * BC)
        hn = _gather_rows_sc(
            node_emb, idx_arr[bsl].reshape(-1)).reshape(BC, 2 * L, D)
        outs.append(_transformer_tc(
            hn, hn, ansf[bsl], nbias, ones_c, Pk, Pv, cdk, cdv, Ak, Av,
            W3b, Aq, (bq * scale).reshape(1, D).astype(bf),
            Wob, bo.reshape(1, D),
            F0wb, F1wb, F0b.reshape(1, D), F1b.reshape(1, D),
            ln2g.reshape(1, D), ln2b.reshape(1, D), ln3g.reshape(1, D),
            ln3b.reshape(1, D), W4b, b4.reshape(1, 1)))
    return jnp.concatenate(outs, axis=0)


# speedup vs baseline: 1.1929x; 1.0051x over previous
import math

import jax
import jax.numpy as jnp
from jax import lax
from jax.experimental import pallas as pl
from jax.experimental.pallas import tpu as pltpu
from jax.experimental.pallas import tpu_sc as plsc

B, L, D, H, V = 1024, 200, 256, 8, 100000
DH = D // H
NW = 32
CHUNK = 200
NC = 4
BC = B // NC


def _gather_rows_sc(table, idx):
    n = idx.shape[0]
    per_w = n // NW
    npairs = per_w // (2 * CHUNK)
    mesh = plsc.VectorSubcoreMesh(core_axis_name="c", subcore_axis_name="s")

    def body(table_ref, idx_ref, out_ref, idx_v, rows_a, rows_b, sem_a,
             sem_b):
        wid = lax.axis_index("s") * 2 + lax.axis_index("c")
        base = wid * per_w
        pltpu.sync_copy(idx_ref.at[pl.ds(base, per_w)], idx_v)

        def gather(off, buf, sem):
            return pltpu.make_async_copy(
                table_ref.at[idx_v.at[pl.ds(off, CHUNK)]], buf, sem)

        gather(0, rows_a, sem_a).start()

        def pair(j, carry):
            o0 = 2 * j * CHUNK
            o1 = o0 + CHUNK
            gather(o1, rows_b, sem_b).start()
            gather(o0, rows_a, sem_a).wait()
            pltpu.sync_copy(rows_a, out_ref.at[pl.ds(base + o0, CHUNK)])

            @pl.when(j + 1 < npairs)
            def _():
                gather(o1 + CHUNK, rows_a, sem_a).start()

            gather(o1, rows_b, sem_b).wait()
            pltpu.sync_copy(rows_b, out_ref.at[pl.ds(base + o1, CHUNK)])
            return carry

        lax.fori_loop(0, npairs, pair, 0)

    fn = pl.kernel(
        body,
        out_type=jax.ShapeDtypeStruct((n, D), jnp.float32),
        mesh=mesh,
        scratch_types=[
            pltpu.VMEM((per_w,), jnp.int32),
            pltpu.VMEM((CHUNK, D), jnp.float32),
            pltpu.VMEM((CHUNK, D), jnp.float32),
            pltpu.SemaphoreType.DMA,
            pltpu.SemaphoreType.DMA,
        ],
    )
    return fn(table, idx)


def _ln_rows(x, g, b):
    m = jnp.mean(x, axis=-1, keepdims=True)
    xc = x - m
    v = jnp.mean(xc * xc, axis=-1, keepdims=True)
    return xc * lax.rsqrt(v + 1e-5) * g + b


BB = 8


def _transformer_body(he_ref, ne_ref, af_ref, nb_ref, ones_ref, Pk_ref,
                      Pv_ref, cdk_ref, cdv_ref, Ak_ref, Av_ref, W3_ref,
                      Aq_ref, bq_ref, Wo_ref, bo_ref, F0w_ref, F1w_ref,
                      F0b_ref, F1b_ref, g2_ref, b2_ref, g3_ref, b3_ref,
                      W4_ref, b4_ref, out_ref):
    bf = jnp.bfloat16
    f32 = jnp.float32

    hef = he_ref[...].reshape(BB * L, D).astype(bf)
    nef = ne_ref[...].reshape(BB * L, D).astype(bf)
    af3 = af_ref[...].astype(bf).reshape(BB, L, 1)
    nb = nb_ref[...]
    ones_c = ones_ref[...]

    k3 = (jnp.dot(hef, Ak_ref[...],
                  preferred_element_type=f32).astype(bf).reshape(BB, L, D)
          + Pk_ref[...][None] + af3 * cdk_ref[...])
    v3 = (jnp.dot(hef, Av_ref[...],
                  preferred_element_type=f32).astype(bf).reshape(BB, L, D)
          + Pv_ref[...][None] + af3 * cdv_ref[...])
    query = jnp.dot(nef, W3_ref[...], preferred_element_type=f32)
    q3 = (jnp.dot(nef, Aq_ref[...],
                  preferred_element_type=f32).astype(bf).reshape(BB, L, D)
          + bq_ref[...])

    ctx_rows = []
    for b in range(BB):
        q2, k2, v2 = q3[b], k3[b], v3[b]
        heads = []
        for h in range(H):
            sl = slice(h * DH, (h + 1) * DH)
            s = lax.dot_general(q2[:, sl], k2[:, sl],
                                (((1,), (1,)), ((), ())),
                                preferred_element_type=f32)
            e = jnp.exp(s).astype(bf) * nb
            vaug = jnp.concatenate([v2[:, sl], ones_c], axis=1)
            o = jnp.dot(e, vaug, preferred_element_type=f32)
            heads.append(o[:, :DH] * (1.0 / o[:, DH:DH + 1]))
        ctx_rows.append(jnp.concatenate(heads, axis=1))
    ctx = jnp.concatenate(ctx_rows, axis=0).astype(bf)

    atn = (jnp.dot(ctx, Wo_ref[...], preferred_element_type=f32)
           + bo_ref[...] + query)
    atn = _ln_rows(atn, g2_ref[...], b2_ref[...])
    hdn = jnp.maximum(
        jnp.dot(atn.astype(bf), F0w_ref[...], preferred_element_type=f32)
        + F0b_ref[...], 0.0).astype(bf)
    ffn = jnp.dot(hdn, F1w_ref[...], preferred_element_type=f32) + F1b_ref[...]
    ffn = _ln_rows(ffn + atn, g3_ref[...], b3_ref[...])
    pred = (jnp.dot(ffn.astype(bf), W4_ref[...], preferred_element_type=f32)
            + b4_ref[...])
    out_ref[...] = pred.reshape(BB, L)


def _transformer_tc(he, ne, ansf, nbias, ones_c, Pk, Pv, cdk, cdv, Ak, Av,
                    W3, Aq, bq, Wo, bo, F0w, F1w, F0b, F1b, g2, b2, g3, b3,
                    W4, b4):
    def blk(shape, imap):
        return pl.BlockSpec(shape, imap)

    row = lambda i: (i, 0, 0)
    row_ne = lambda i: (i, 1, 0)
    const2 = lambda i: (0, 0)
    in_specs = [
        blk((BB, L, D), row),
        blk((BB, L, D), row_ne),
        blk((BB, L), lambda i: (i, 0)),
        blk((L, L), const2),
        blk((L, 1), const2),
        blk((L, D), const2),
        blk((L, D), const2),
        blk((1, D), const2),
        blk((1, D), const2),
        blk((D, D), const2),
        blk((D, D), const2),
        blk((D, D), const2),
        blk((D, D), const2),
        blk((1, D), const2),
        blk((D, D), const2),
        blk((1, D), const2),
        blk((D, D), const2),
        blk((D, D), const2),
        blk((1, D), const2),
        blk((1, D), const2),
        blk((1, D), const2),
        blk((1, D), const2),
        blk((1, D), const2),
        blk((1, D), const2),
        blk((D, 1), const2),
        blk((1, 1), const2),
    ]
    out = pl.pallas_call(
        _transformer_body,
        grid=(BC // BB,),
        in_specs=in_specs,
        out_specs=pl.BlockSpec((BB, L), lambda i: (i, 0)),
        out_shape=jax.ShapeDtypeStruct((BC, L), jnp.float32),
        compiler_params=pltpu.CompilerParams(
            dimension_semantics=("arbitrary",)),
    )(he, ne, ansf, nbias, ones_c, Pk, Pv, cdk, cdv, Ak, Av, W3, Aq, bq, Wo,
      bo, F0w, F1w, F0b, F1b, g2, b2, g3, b3, W4, b4)
    return out


def kernel(hist_seq, hist_answers, new_seq, node_emb, corr_emb, pos_emb, W0,
           b0, W1, W2, W3, Wq, bq, Wk, bk, Wv, bv, Wo, bo, F0w, F0b, F1w,
           F1b, ln2g, ln2b, ln3g, ln3b, W4, b4):
    idx_arr = jnp.concatenate([hist_seq, new_seq], axis=1).astype(jnp.int32)

    bf = jnp.bfloat16
    corrW = corr_emb @ W0[D:]
    P = pos_emb + b0[None, :] + corrW[0][None, :]
    cdelta = (corrW[1] - corrW[0]).reshape(1, D)
    scale = jnp.float32(1.0 / math.sqrt(DH))
    cols_gt_rows = (jnp.arange(L)[None, :] > jnp.arange(L)[:, None])
    nbias = jnp.where(cols_gt_rows, 0.0, 1.0).astype(bf)
    ones_c = jnp.ones((L, 1), bf)
    Wkf = W2 @ Wk
    Wvf = W1 @ Wv
    Ak = (W0[:D] @ Wkf).astype(bf)
    Av = (W0[:D] @ Wvf).astype(bf)
    Pk = (P @ Wkf + bk[None, :]).astype(bf)
    Pv = (P @ Wvf + bv[None, :]).astype(bf)
    cdk = (cdelta @ Wkf).astype(bf)
    cdv = (cdelta @ Wvf).astype(bf)
    W3b = W3.astype(bf)
    Aq = ((W3 @ Wq) * scale).astype(bf)
    Wob = Wo.astype(bf)
    F0wb = F0w.astype(bf)
    F1wb = F1w.astype(bf)
    W4b = W4.astype(bf)
    ansf = hist_answers

    outs = []
    for c in range(NC):
        bsl = slice(c * BC, (c + 1) * BC)
        hn = _gather_rows_sc(
            node_emb, idx_arr[bsl].reshape(-1)).reshape(BC, 2 * L, D)
        outs.append(_transformer_tc(
            hn, hn, ansf[bsl], nbias, ones_c, Pk, Pv, cdk, cdv, Ak, Av,
            W3b, Aq, (bq * scale).reshape(1, D).astype(bf),
            Wob, bo.reshape(1, D),
            F0wb, F1wb, F0b.reshape(1, D), F1b.reshape(1, D),
            ln2g.reshape(1, D), ln2b.reshape(1, D), ln3g.reshape(1, D),
            ln3b.reshape(1, D), W4b, b4.reshape(1, 1)))
    return jnp.concatenate(outs, axis=0)


# speedup vs baseline: 1.3255x; 1.1111x over previous
import math

import jax
import jax.numpy as jnp
from jax import lax
from jax.experimental import pallas as pl
from jax.experimental.pallas import tpu as pltpu
from jax.experimental.pallas import tpu_sc as plsc

B, L, D, H, V = 1024, 200, 256, 8, 100000
DH = D // H
NW = 32
CHUNK = 200
NC = 4
BC = B // NC


def _gather_rows_sc(table, idx):
    n = idx.shape[0]
    per_w = n // NW
    npairs = per_w // (2 * CHUNK)
    mesh = plsc.VectorSubcoreMesh(core_axis_name="c", subcore_axis_name="s")

    def body(table_ref, idx_ref, out_ref, idx_v, rows_a, rows_b, sem_a,
             sem_b):
        wid = lax.axis_index("s") * 2 + lax.axis_index("c")
        base = wid * per_w
        pltpu.sync_copy(idx_ref.at[pl.ds(base, per_w)], idx_v)

        def gather(off, buf, sem):
            return pltpu.make_async_copy(
                table_ref.at[idx_v.at[pl.ds(off, CHUNK)]], buf, sem)

        gather(0, rows_a, sem_a).start()

        def pair(j, carry):
            o0 = 2 * j * CHUNK
            o1 = o0 + CHUNK
            gather(o1, rows_b, sem_b).start()
            gather(o0, rows_a, sem_a).wait()
            pltpu.sync_copy(rows_a, out_ref.at[pl.ds(base + o0, CHUNK)])

            @pl.when(j + 1 < npairs)
            def _():
                gather(o1 + CHUNK, rows_a, sem_a).start()

            gather(o1, rows_b, sem_b).wait()
            pltpu.sync_copy(rows_b, out_ref.at[pl.ds(base + o1, CHUNK)])
            return carry

        lax.fori_loop(0, npairs, pair, 0)

    fn = pl.kernel(
        body,
        out_type=jax.ShapeDtypeStruct((n, D), jnp.float32),
        mesh=mesh,
        scratch_types=[
            pltpu.VMEM((per_w,), jnp.int32),
            pltpu.VMEM((CHUNK, D), jnp.float32),
            pltpu.VMEM((CHUNK, D), jnp.float32),
            pltpu.SemaphoreType.DMA,
            pltpu.SemaphoreType.DMA,
        ],
    )
    return fn(table, idx)


def _ln_rows(x, g, b):
    m = jnp.mean(x, axis=-1, keepdims=True)
    xc = x - m
    v = jnp.mean(xc * xc, axis=-1, keepdims=True)
    return xc * lax.rsqrt(v + 1e-5) * g + b


BB = 16


def _transformer_body(he_ref, ne_ref, af_ref, nb_ref, ones_ref, Pk_ref,
                      Pv_ref, cdk_ref, cdv_ref, Ak_ref, Av_ref, W3_ref,
                      Aq_ref, bq_ref, Wo_ref, bo_ref, F0w_ref, F1w_ref,
                      F0b_ref, F1b_ref, g2_ref, b2_ref, g3_ref, b3_ref,
                      W4_ref, b4_ref, out_ref):
    bf = jnp.bfloat16
    f32 = jnp.float32

    hef = he_ref[...].reshape(BB * L, D).astype(bf)
    nef = ne_ref[...].reshape(BB * L, D).astype(bf)
    af3 = af_ref[...].astype(bf).reshape(BB, L, 1)
    nb = nb_ref[...]
    ones_c = ones_ref[...]

    k3 = (jnp.dot(hef, Ak_ref[...],
                  preferred_element_type=f32).astype(bf).reshape(BB, L, D)
          + Pk_ref[...][None] + af3 * cdk_ref[...])
    v3 = (jnp.dot(hef, Av_ref[...],
                  preferred_element_type=f32).astype(bf).reshape(BB, L, D)
          + Pv_ref[...][None] + af3 * cdv_ref[...])
    query = jnp.dot(nef, W3_ref[...], preferred_element_type=f32)
    q3 = (jnp.dot(nef, Aq_ref[...],
                  preferred_element_type=f32).astype(bf).reshape(BB, L, D)
          + bq_ref[...])

    ctx_rows = []
    for b in range(BB):
        q2, k2, v2 = q3[b], k3[b], v3[b]
        heads = []
        for h in range(H):
            sl = slice(h * DH, (h + 1) * DH)
            s = lax.dot_general(q2[:, sl], k2[:, sl],
                                (((1,), (1,)), ((), ())),
                                preferred_element_type=f32)
            e = jnp.exp(s).astype(bf) * nb
            vaug = jnp.concatenate([v2[:, sl], ones_c], axis=1)
            o = jnp.dot(e, vaug, preferred_element_type=f32)
            heads.append(o[:, :DH] * (1.0 / o[:, DH:DH + 1]))
        ctx_rows.append(jnp.concatenate(heads, axis=1))
    ctx = jnp.concatenate(ctx_rows, axis=0).astype(bf)

    atn = (jnp.dot(ctx, Wo_ref[...], preferred_element_type=f32)
           + bo_ref[...] + query)
    atn = _ln_rows(atn, g2_ref[...], b2_ref[...])
    hdn = jnp.maximum(
        jnp.dot(atn.astype(bf), F0w_ref[...], preferred_element_type=f32)
        + F0b_ref[...], 0.0).astype(bf)
    ffn = jnp.dot(hdn, F1w_ref[...], preferred_element_type=f32) + F1b_ref[...]
    ffn = _ln_rows(ffn + atn, g3_ref[...], b3_ref[...])
    pred = (jnp.dot(ffn.astype(bf), W4_ref[...], preferred_element_type=f32)
            + b4_ref[...])
    out_ref[...] = pred.reshape(BB, L)


def _transformer_tc(he, ne, ansf, nbias, ones_c, Pk, Pv, cdk, cdv, Ak, Av,
                    W3, Aq, bq, Wo, bo, F0w, F1w, F0b, F1b, g2, b2, g3, b3,
                    W4, b4):
    def blk(shape, imap):
        return pl.BlockSpec(shape, imap)

    row = lambda i: (i, 0, 0)
    row_ne = lambda i: (i, 1, 0)
    const2 = lambda i: (0, 0)
    in_specs = [
        blk((BB, L, D), row),
        blk((BB, L, D), row_ne),
        blk((BB, L), lambda i: (i, 0)),
        blk((L, L), const2),
        blk((L, 1), const2),
        blk((L, D), const2),
        blk((L, D), const2),
        blk((1, D), const2),
        blk((1, D), const2),
        blk((D, D), const2),
        blk((D, D), const2),
        blk((D, D), const2),
        blk((D, D), const2),
        blk((1, D), const2),
        blk((D, D), const2),
        blk((1, D), const2),
        blk((D, D), const2),
        blk((D, D), const2),
        blk((1, D), const2),
        blk((1, D), const2),
        blk((1, D), const2),
        blk((1, D), const2),
        blk((1, D), const2),
        blk((1, D), const2),
        blk((D, 1), const2),
        blk((1, 1), const2),
    ]
    out = pl.pallas_call(
        _transformer_body,
        grid=(BC // BB,),
        in_specs=in_specs,
        out_specs=pl.BlockSpec((BB, L), lambda i: (i, 0)),
        out_shape=jax.ShapeDtypeStruct((BC, L), jnp.float32),
        compiler_params=pltpu.CompilerParams(
            dimension_semantics=("arbitrary",)),
    )(he, ne, ansf, nbias, ones_c, Pk, Pv, cdk, cdv, Ak, Av, W3, Aq, bq, Wo,
      bo, F0w, F1w, F0b, F1b, g2, b2, g3, b3, W4, b4)
    return out


def kernel(hist_seq, hist_answers, new_seq, node_emb, corr_emb, pos_emb, W0,
           b0, W1, W2, W3, Wq, bq, Wk, bk, Wv, bv, Wo, bo, F0w, F0b, F1w,
           F1b, ln2g, ln2b, ln3g, ln3b, W4, b4):
    idx_arr = jnp.concatenate([hist_seq, new_seq], axis=1).astype(jnp.int32)

    bf = jnp.bfloat16
    corrW = corr_emb @ W0[D:]
    P = pos_emb + b0[None, :] + corrW[0][None, :]
    cdelta = (corrW[1] - corrW[0]).reshape(1, D)
    scale = jnp.float32(1.0 / math.sqrt(DH))
    cols_gt_rows = (jnp.arange(L)[None, :] > jnp.arange(L)[:, None])
    nbias = jnp.where(cols_gt_rows, 0.0, 1.0).astype(bf)
    ones_c = jnp.ones((L, 1), bf)
    Wkf = W2 @ Wk
    Wvf = W1 @ Wv
    Ak = (W0[:D] @ Wkf).astype(bf)
    Av = (W0[:D] @ Wvf).astype(bf)
    Pk = (P @ Wkf + bk[None, :]).astype(bf)
    Pv = (P @ Wvf + bv[None, :]).astype(bf)
    cdk = (cdelta @ Wkf).astype(bf)
    cdv = (cdelta @ Wvf).astype(bf)
    W3b = W3.astype(bf)
    Aq = ((W3 @ Wq) * scale).astype(bf)
    Wob = Wo.astype(bf)
    F0wb = F0w.astype(bf)
    F1wb = F1w.astype(bf)
    W4b = W4.astype(bf)
    ansf = hist_answers

    outs = []
    for c in range(NC):
        bsl = slice(c * BC, (c + 1) * BC)
        hn = _gather_rows_sc(
            node_emb, idx_arr[bsl].reshape(-1)).reshape(BC, 2 * L, D)
        outs.append(_transformer_tc(
            hn, hn, ansf[bsl], nbias, ones_c, Pk, Pv, cdk, cdv, Ak, Av,
            W3b, Aq, (bq * scale).reshape(1, D).astype(bf),
            Wob, bo.reshape(1, D),
            F0wb, F1wb, F0b.reshape(1, D), F1b.reshape(1, D),
            ln2g.reshape(1, D), ln2b.reshape(1, D), ln3g.reshape(1, D),
            ln3b.reshape(1, D), W4b, b4.reshape(1, 1)))
    return jnp.concatenate(outs, axis=0)


# speedup vs baseline: 1.3620x; 1.0276x over previous
import math

import jax
import jax.numpy as jnp
from jax import lax
from jax.experimental import pallas as pl
from jax.experimental.pallas import tpu as pltpu
from jax.experimental.pallas import tpu_sc as plsc

B, L, D, H, V = 1024, 200, 256, 8, 100000
DH = D // H
NW = 32
CHUNK = 200
NC = 4
BC = B // NC


def _gather_rows_sc(table, idx):
    n = idx.shape[0]
    per_w = n // NW
    npairs = per_w // (2 * CHUNK)
    mesh = plsc.VectorSubcoreMesh(core_axis_name="c", subcore_axis_name="s")

    def body(table_ref, idx_ref, out_ref, idx_v, rows_a, rows_b, sem_a,
             sem_b):
        wid = lax.axis_index("s") * 2 + lax.axis_index("c")
        base = wid * per_w
        pltpu.sync_copy(idx_ref.at[pl.ds(base, per_w)], idx_v)

        def gather(off, buf, sem):
            return pltpu.make_async_copy(
                table_ref.at[idx_v.at[pl.ds(off, CHUNK)]], buf, sem)

        gather(0, rows_a, sem_a).start()

        def pair(j, carry):
            o0 = 2 * j * CHUNK
            o1 = o0 + CHUNK
            gather(o1, rows_b, sem_b).start()
            gather(o0, rows_a, sem_a).wait()
            pltpu.sync_copy(rows_a, out_ref.at[pl.ds(base + o0, CHUNK)])

            @pl.when(j + 1 < npairs)
            def _():
                gather(o1 + CHUNK, rows_a, sem_a).start()

            gather(o1, rows_b, sem_b).wait()
            pltpu.sync_copy(rows_b, out_ref.at[pl.ds(base + o1, CHUNK)])
            return carry

        lax.fori_loop(0, npairs, pair, 0)

    fn = pl.kernel(
        body,
        out_type=jax.ShapeDtypeStruct((n, D), jnp.float32),
        mesh=mesh,
        scratch_types=[
            pltpu.VMEM((per_w,), jnp.int32),
            pltpu.VMEM((CHUNK, D), jnp.float32),
            pltpu.VMEM((CHUNK, D), jnp.float32),
            pltpu.SemaphoreType.DMA,
            pltpu.SemaphoreType.DMA,
        ],
    )
    return fn(table, idx)


def _ln_rows(x, g, b):
    m = jnp.mean(x, axis=-1, keepdims=True)
    xc = x - m
    v = jnp.mean(xc * xc, axis=-1, keepdims=True)
    return xc * lax.rsqrt(v + 1e-5) * g + b


BB = 32


def _transformer_body(he_ref, ne_ref, af_ref, nb_ref, ones_ref, Pk_ref,
                      Pv_ref, cdk_ref, cdv_ref, Ak_ref, Av_ref, W3_ref,
                      Aq_ref, bq_ref, Wo_ref, bo_ref, F0w_ref, F1w_ref,
                      F0b_ref, F1b_ref, g2_ref, b2_ref, g3_ref, b3_ref,
                      W4_ref, b4_ref, out_ref):
    bf = jnp.bfloat16
    f32 = jnp.float32

    hef = he_ref[...].reshape(BB * L, D).astype(bf)
    nef = ne_ref[...].reshape(BB * L, D).astype(bf)
    af3 = af_ref[...].astype(bf).reshape(BB, L, 1)
    nb = nb_ref[...]
    ones_c = ones_ref[...]

    k3 = (jnp.dot(hef, Ak_ref[...],
                  preferred_element_type=f32).astype(bf).reshape(BB, L, D)
          + Pk_ref[...][None] + af3 * cdk_ref[...])
    v3 = (jnp.dot(hef, Av_ref[...],
                  preferred_element_type=f32).astype(bf).reshape(BB, L, D)
          + Pv_ref[...][None] + af3 * cdv_ref[...])
    query = jnp.dot(nef, W3_ref[...], preferred_element_type=f32)
    q3 = (jnp.dot(nef, Aq_ref[...],
                  preferred_element_type=f32).astype(bf).reshape(BB, L, D)
          + bq_ref[...])

    ctx_rows = []
    for b in range(BB):
        q2, k2, v2 = q3[b], k3[b], v3[b]
        heads = []
        for h in range(H):
            sl = slice(h * DH, (h + 1) * DH)
            s = lax.dot_general(q2[:, sl], k2[:, sl],
                                (((1,), (1,)), ((), ())),
                                preferred_element_type=f32)
            e = jnp.exp(s).astype(bf) * nb
            vaug = jnp.concatenate([v2[:, sl], ones_c], axis=1)
            o = jnp.dot(e, vaug, preferred_element_type=f32)
            heads.append(o[:, :DH] * (1.0 / o[:, DH:DH + 1]))
        ctx_rows.append(jnp.concatenate(heads, axis=1))
    ctx = jnp.concatenate(ctx_rows, axis=0).astype(bf)

    atn = (jnp.dot(ctx, Wo_ref[...], preferred_element_type=f32)
           + bo_ref[...] + query)
    atn = _ln_rows(atn, g2_ref[...], b2_ref[...])
    hdn = jnp.maximum(
        jnp.dot(atn.astype(bf), F0w_ref[...], preferred_element_type=f32)
        + F0b_ref[...], 0.0).astype(bf)
    ffn = jnp.dot(hdn, F1w_ref[...], preferred_element_type=f32) + F1b_ref[...]
    ffn = _ln_rows(ffn + atn, g3_ref[...], b3_ref[...])
    pred = (jnp.dot(ffn.astype(bf), W4_ref[...], preferred_element_type=f32)
            + b4_ref[...])
    out_ref[...] = pred.reshape(BB, L)


def _transformer_tc(he, ne, ansf, nbias, ones_c, Pk, Pv, cdk, cdv, Ak, Av,
                    W3, Aq, bq, Wo, bo, F0w, F1w, F0b, F1b, g2, b2, g3, b3,
                    W4, b4):
    def blk(shape, imap):
        return pl.BlockSpec(shape, imap)

    row = lambda i: (i, 0, 0)
    row_ne = lambda i: (i, 1, 0)
    const2 = lambda i: (0, 0)
    in_specs = [
        blk((BB, L, D), row),
        blk((BB, L, D), row_ne),
        blk((BB, L), lambda i: (i, 0)),
        blk((L, L), const2),
        blk((L, 1), const2),
        blk((L, D), const2),
        blk((L, D), const2),
        blk((1, D), const2),
        blk((1, D), const2),
        blk((D, D), const2),
        blk((D, D), const2),
        blk((D, D), const2),
        blk((D, D), const2),
        blk((1, D), const2),
        blk((D, D), const2),
        blk((1, D), const2),
        blk((D, D), const2),
        blk((D, D), const2),
        blk((1, D), const2),
        blk((1, D), const2),
        blk((1, D), const2),
        blk((1, D), const2),
        blk((1, D), const2),
        blk((1, D), const2),
        blk((D, 1), const2),
        blk((1, 1), const2),
    ]
    out = pl.pallas_call(
        _transformer_body,
        grid=(BC // BB,),
        in_specs=in_specs,
        out_specs=pl.BlockSpec((BB, L), lambda i: (i, 0)),
        out_shape=jax.ShapeDtypeStruct((BC, L), jnp.float32),
        compiler_params=pltpu.CompilerParams(
            dimension_semantics=("arbitrary",)),
    )(he, ne, ansf, nbias, ones_c, Pk, Pv, cdk, cdv, Ak, Av, W3, Aq, bq, Wo,
      bo, F0w, F1w, F0b, F1b, g2, b2, g3, b3, W4, b4)
    return out


def kernel(hist_seq, hist_answers, new_seq, node_emb, corr_emb, pos_emb, W0,
           b0, W1, W2, W3, Wq, bq, Wk, bk, Wv, bv, Wo, bo, F0w, F0b, F1w,
           F1b, ln2g, ln2b, ln3g, ln3b, W4, b4):
    idx_arr = jnp.concatenate([hist_seq, new_seq], axis=1).astype(jnp.int32)

    bf = jnp.bfloat16
    corrW = corr_emb @ W0[D:]
    P = pos_emb + b0[None, :] + corrW[0][None, :]
    cdelta = (corrW[1] - corrW[0]).reshape(1, D)
    scale = jnp.float32(1.0 / math.sqrt(DH))
    cols_gt_rows = (jnp.arange(L)[None, :] > jnp.arange(L)[:, None])
    nbias = jnp.where(cols_gt_rows, 0.0, 1.0).astype(bf)
    ones_c = jnp.ones((L, 1), bf)
    Wkf = W2 @ Wk
    Wvf = W1 @ Wv
    Ak = (W0[:D] @ Wkf).astype(bf)
    Av = (W0[:D] @ Wvf).astype(bf)
    Pk = (P @ Wkf + bk[None, :]).astype(bf)
    Pv = (P @ Wvf + bv[None, :]).astype(bf)
    cdk = (cdelta @ Wkf).astype(bf)
    cdv = (cdelta @ Wvf).astype(bf)
    W3b = W3.astype(bf)
    Aq = ((W3 @ Wq) * scale).astype(bf)
    Wob = Wo.astype(bf)
    F0wb = F0w.astype(bf)
    F1wb = F1w.astype(bf)
    W4b = W4.astype(bf)
    ansf = hist_answers

    outs = []
    for c in range(NC):
        bsl = slice(c * BC, (c + 1) * BC)
        hn = _gather_rows_sc(
            node_emb, idx_arr[bsl].reshape(-1)).reshape(BC, 2 * L, D)
        outs.append(_transformer_tc(
            hn, hn, ansf[bsl], nbias, ones_c, Pk, Pv, cdk, cdv, Ak, Av,
            W3b, Aq, (bq * scale).reshape(1, D).astype(bf),
            Wob, bo.reshape(1, D),
            F0wb, F1wb, F0b.reshape(1, D), F1b.reshape(1, D),
            ln2g.reshape(1, D), ln2b.reshape(1, D), ln3g.reshape(1, D),
            ln3b.reshape(1, D), W4b, b4.reshape(1, 1)))
    return jnp.concatenate(outs, axis=0)
